# Initial kernel scaffold; baseline (speedup 1.0000x reference)
import functools

import jax
import jax.numpy as jnp
from jax import lax
from jax.experimental import pallas as pl

_N_GROUPS = 2
_N_CODES = 1024
_C = 512
_DIM = _C // _N_GROUPS
_LAYERS = 2
_T_BLK = 512


def _vq_kernel(x_ref, cb_ref, q_ref, idx_ref, loss_ref, *, nel):
    b = pl.program_id(0)
    t = pl.program_id(1)

    @pl.when(jnp.logical_and(b == 0, t == 0))
    def _():
        loss_ref[0, 0] = 0.0

    x = x_ref[0]
    res = x
    q_acc = jnp.zeros_like(x)
    idx_rows = []
    loss_sum = 0.0
    for layer in range(_LAYERS):
        zq_parts = []
        for g in range(_N_GROUPS):
            xg = res[g * _DIM:(g + 1) * _DIM, :]
            cb = cb_ref[layer, g]
            x2 = jnp.sum(xg * xg, axis=0, keepdims=True)
            c2 = jnp.sum(cb * cb, axis=1, keepdims=True)
            m = lax.dot_general(cb, xg, (((1,), (0,)), ((), ())),
                                preferred_element_type=jnp.float32)
            d = (x2 + c2) - 2.0 * m
            dmin = jnp.min(d, axis=0, keepdims=True)
            rows = lax.broadcasted_iota(jnp.int32, d.shape, 0)
            hit = d == dmin
            idx = jnp.min(jnp.where(hit, rows, _N_CODES), axis=0,
                          keepdims=True)
            oh = (rows == idx).astype(jnp.float32)
            zq = lax.dot_general(cb, oh, (((0,), (0,)), ((), ())),
                                 preferred_element_type=jnp.float32,
                                 precision=lax.Precision.HIGHEST)
            zq_parts.append(zq)
            idx_rows.append(idx)
        zq_full = jnp.concatenate(zq_parts, axis=0)
        diff = zq_full - res
        loss_sum = loss_sum + jnp.sum(diff * diff)
        res = res - zq_full
        q_acc = q_acc + zq_full
    q_ref[0] = q_acc
    pad = jnp.zeros((8 - 2 * _LAYERS, x.shape[1]), jnp.int32)
    idx_ref[...] = jnp.concatenate(idx_rows + [pad], axis=0)
    loss_ref[0, 0] += (0.625 / nel) * loss_sum


def kernel(xin, codebooks):
    B, C, T = xin.shape
    n_tblk = T // _T_BLK
    nel = B * C * T
    grid = (B, n_tblk)
    q, idx8, loss = pl.pallas_call(
        functools.partial(_vq_kernel, nel=nel),
        grid=grid,
        in_specs=[
            pl.BlockSpec((1, C, _T_BLK), lambda b, t: (b, 0, t)),
            pl.BlockSpec((_LAYERS, _N_GROUPS, _N_CODES, _DIM),
                         lambda b, t: (0, 0, 0, 0)),
        ],
        out_specs=[
            pl.BlockSpec((1, C, _T_BLK), lambda b, t: (b, 0, t)),
            pl.BlockSpec((8, _T_BLK), lambda b, t: (0, b * n_tblk + t)),
            pl.BlockSpec((1, 1), lambda b, t: (0, 0)),
        ],
        out_shape=[
            jax.ShapeDtypeStruct((B, C, T), jnp.float32),
            jax.ShapeDtypeStruct((8, B * T), jnp.int32),
            jax.ShapeDtypeStruct((1, 1), jnp.float32),
        ],
    )(xin, codebooks)
    return q, loss.reshape(()), idx8[:2 * _LAYERS]


# speedup vs baseline: 1.7601x; 1.7601x over previous
import functools

import jax
import jax.numpy as jnp
from jax import lax
from jax.experimental import pallas as pl

_N_GROUPS = 2
_N_CODES = 1024
_C = 512
_DIM = _C // _N_GROUPS
_LAYERS = 2
_T_BLK = 512


def _vq_kernel(x_ref, cb_ref, q_ref, idx_ref, loss_ref, *, nel):
    b = pl.program_id(0)
    t = pl.program_id(1)

    @pl.when(jnp.logical_and(b == 0, t == 0))
    def _():
        loss_ref[...] = jnp.zeros((1, 1), jnp.float32)

    x = x_ref[0]
    res = x
    q_acc = jnp.zeros_like(x)
    idx_rows = []
    loss_sum = 0.0
    for layer in range(_LAYERS):
        zq_parts = []
        for g in range(_N_GROUPS):
            xg = res[g * _DIM:(g + 1) * _DIM, :]
            cb = cb_ref[layer, g]
            x2 = jnp.sum(xg * xg, axis=0, keepdims=True)
            c2 = jnp.sum(cb * cb, axis=1, keepdims=True)
            m = lax.dot_general(cb, xg, (((1,), (0,)), ((), ())),
                                preferred_element_type=jnp.float32)
            d = (x2 + c2) - 2.0 * m
            dmin = jnp.min(d, axis=0, keepdims=True)
            rows = lax.broadcasted_iota(jnp.int32, d.shape, 0)
            hit = d == dmin
            idx = jnp.min(jnp.where(hit, rows, _N_CODES), axis=0,
                          keepdims=True)
            oh = (rows == idx).astype(jnp.float32)
            zq = lax.dot_general(cb, oh, (((0,), (0,)), ((), ())),
                                 preferred_element_type=jnp.float32,
                                 precision=lax.Precision.HIGHEST)
            zq_parts.append(zq)
            idx_rows.append(idx)
        zq_full = jnp.concatenate(zq_parts, axis=0)
        diff = zq_full - res
        loss_sum = loss_sum + jnp.sum(diff * diff)
        res = res - zq_full
        q_acc = q_acc + zq_full
    q_ref[0] = q_acc
    pad = jnp.zeros((8 - 2 * _LAYERS, x.shape[1]), jnp.int32)
    idx_ref[...] = jnp.concatenate(idx_rows + [pad], axis=0)
    loss_ref[...] += ((0.625 / nel) * loss_sum).reshape(1, 1)


def kernel(xin, codebooks):
    B, C, T = xin.shape
    n_tblk = T // _T_BLK
    nel = B * C * T
    grid = (B, n_tblk)
    q, idx8, loss = pl.pallas_call(
        functools.partial(_vq_kernel, nel=nel),
        grid=grid,
        in_specs=[
            pl.BlockSpec((1, C, _T_BLK), lambda b, t: (b, 0, t)),
            pl.BlockSpec((_LAYERS, _N_GROUPS, _N_CODES, _DIM),
                         lambda b, t: (0, 0, 0, 0)),
        ],
        out_specs=[
            pl.BlockSpec((1, C, _T_BLK), lambda b, t: (b, 0, t)),
            pl.BlockSpec((8, _T_BLK), lambda b, t: (0, b * n_tblk + t)),
            pl.BlockSpec((1, 1), lambda b, t: (0, 0)),
        ],
        out_shape=[
            jax.ShapeDtypeStruct((B, C, T), jnp.float32),
            jax.ShapeDtypeStruct((8, B * T), jnp.int32),
            jax.ShapeDtypeStruct((1, 1), jnp.float32),
        ],
    )(xin, codebooks)
    return q, loss.reshape(()), idx8[:2 * _LAYERS]


# speedup vs baseline: 3.0842x; 1.7523x over previous
import functools

import jax
import jax.numpy as jnp
from jax import lax
from jax.experimental import pallas as pl

_N_GROUPS = 2
_N_CODES = 1024
_C = 512
_DIM = _C // _N_GROUPS
_LAYERS = 2
_T_BLK = 512


def _vq_kernel(x_ref, cb_ref, cbh_ref, cbl_ref, q_ref, idx_ref, loss_ref, *,
               nel):
    b = pl.program_id(0)
    t = pl.program_id(1)

    @pl.when(jnp.logical_and(b == 0, t == 0))
    def _():
        loss_ref[...] = jnp.zeros((1, 1), jnp.float32)

    x = x_ref[0]
    res = x
    q_acc = jnp.zeros_like(x)
    idx_rows = []
    loss_sum = 0.0
    for layer in range(_LAYERS):
        zq_parts = []
        for g in range(_N_GROUPS):
            xg = res[g * _DIM:(g + 1) * _DIM, :]
            cb = cb_ref[layer, g]
            x2 = jnp.sum(xg * xg, axis=0, keepdims=True)
            c2 = jnp.sum(cb * cb, axis=1, keepdims=True)
            m = lax.dot_general(cb, xg, (((1,), (0,)), ((), ())),
                                preferred_element_type=jnp.float32)
            d = (x2 + c2) - 2.0 * m
            dmin = jnp.min(d, axis=0, keepdims=True)
            rows = lax.broadcasted_iota(jnp.int32, d.shape, 0)
            hit = d == dmin
            idx = jnp.min(jnp.where(hit, rows, _N_CODES), axis=0,
                          keepdims=True)
            oh = (rows == idx).astype(jnp.bfloat16)
            dn = (((0,), (0,)), ((), ()))
            zq = (lax.dot_general(cbh_ref[layer, g], oh, dn,
                                  preferred_element_type=jnp.float32)
                  + lax.dot_general(cbl_ref[layer, g], oh, dn,
                                    preferred_element_type=jnp.float32))
            zq_parts.append(zq)
            idx_rows.append(idx)
        zq_full = jnp.concatenate(zq_parts, axis=0)
        diff = zq_full - res
        loss_sum = loss_sum + jnp.sum(diff * diff)
        res = res - zq_full
        q_acc = q_acc + zq_full
    q_ref[0] = q_acc
    pad = jnp.zeros((8 - 2 * _LAYERS, x.shape[1]), jnp.int32)
    idx_ref[...] = jnp.concatenate(idx_rows + [pad], axis=0)
    loss_ref[...] += ((0.625 / nel) * loss_sum).reshape(1, 1)


def kernel(xin, codebooks):
    B, C, T = xin.shape
    n_tblk = T // _T_BLK
    nel = B * C * T
    grid = (B, n_tblk)
    cb_hi = codebooks.astype(jnp.bfloat16)
    cb_lo = (codebooks - cb_hi.astype(jnp.float32)).astype(jnp.bfloat16)
    cb_spec = pl.BlockSpec((_LAYERS, _N_GROUPS, _N_CODES, _DIM),
                           lambda b, t: (0, 0, 0, 0))
    q, idx8, loss = pl.pallas_call(
        functools.partial(_vq_kernel, nel=nel),
        grid=grid,
        in_specs=[
            pl.BlockSpec((1, C, _T_BLK), lambda b, t: (b, 0, t)),
            cb_spec, cb_spec, cb_spec,
        ],
        out_specs=[
            pl.BlockSpec((1, C, _T_BLK), lambda b, t: (b, 0, t)),
            pl.BlockSpec((8, _T_BLK), lambda b, t: (0, b * n_tblk + t)),
            pl.BlockSpec((1, 1), lambda b, t: (0, 0)),
        ],
        out_shape=[
            jax.ShapeDtypeStruct((B, C, T), jnp.float32),
            jax.ShapeDtypeStruct((8, B * T), jnp.int32),
            jax.ShapeDtypeStruct((1, 1), jnp.float32),
        ],
    )(xin, codebooks, cb_hi, cb_lo)
    return q, loss.reshape(()), idx8[:2 * _LAYERS]


# speedup vs baseline: 3.1230x; 1.0126x over previous
import functools

import jax
import jax.numpy as jnp
from jax import lax
from jax.experimental import pallas as pl

_N_GROUPS = 2
_N_CODES = 1024
_C = 512
_DIM = _C // _N_GROUPS
_LAYERS = 2
_T_BLK = 512


def _vq_kernel(x_ref, cb_ref, cbh_ref, cbl_ref, q_ref, idx_ref, loss_ref, *,
               nel):
    b = pl.program_id(0)
    t = pl.program_id(1)

    @pl.when(jnp.logical_and(b == 0, t == 0))
    def _():
        loss_ref[...] = jnp.zeros((1, 1), jnp.float32)

    x = x_ref[0]
    res = x
    q_acc = jnp.zeros_like(x)
    idx_rows = []
    loss_sum = 0.0
    for layer in range(_LAYERS):
        zq_parts = []
        for g in range(_N_GROUPS):
            xg = res[g * _DIM:(g + 1) * _DIM, :]
            cb = cb_ref[layer, g]
            x2 = jnp.sum(xg * xg, axis=0, keepdims=True)
            c2 = jnp.sum(cb * cb, axis=1, keepdims=True)
            m = lax.dot_general(cb, xg, (((1,), (0,)), ((), ())),
                                preferred_element_type=jnp.float32)
            d = (x2 + c2) - 2.0 * m
            dmin = jnp.min(d, axis=0, keepdims=True)
            rows = lax.broadcasted_iota(jnp.int32, d.shape, 0)
            hit = d == dmin
            idx = jnp.min(jnp.where(hit, rows, _N_CODES), axis=0,
                          keepdims=True)
            oh = (rows == idx).astype(jnp.bfloat16)
            dn = (((0,), (0,)), ((), ()))
            zq = (lax.dot_general(cbh_ref[layer, g], oh, dn,
                                  preferred_element_type=jnp.float32)
                  + lax.dot_general(cbl_ref[layer, g], oh, dn,
                                    preferred_element_type=jnp.float32))
            zq_parts.append(zq)
            idx_rows.append(idx)
            loss_sum = loss_sum + jnp.sum(dmin)
        zq_full = jnp.concatenate(zq_parts, axis=0)
        res = res - zq_full
        q_acc = q_acc + zq_full
    q_ref[0] = q_acc
    pad = jnp.zeros((8 - 2 * _LAYERS, x.shape[1]), jnp.int32)
    idx_ref[...] = jnp.concatenate(idx_rows + [pad], axis=0)
    loss_ref[...] += ((0.625 / nel) * loss_sum).reshape(1, 1)


def kernel(xin, codebooks):
    B, C, T = xin.shape
    n_tblk = T // _T_BLK
    nel = B * C * T
    grid = (B, n_tblk)
    cb_hi = codebooks.astype(jnp.bfloat16)
    cb_lo = (codebooks - cb_hi.astype(jnp.float32)).astype(jnp.bfloat16)
    cb_spec = pl.BlockSpec((_LAYERS, _N_GROUPS, _N_CODES, _DIM),
                           lambda b, t: (0, 0, 0, 0))
    q, idx8, loss = pl.pallas_call(
        functools.partial(_vq_kernel, nel=nel),
        grid=grid,
        in_specs=[
            pl.BlockSpec((1, C, _T_BLK), lambda b, t: (b, 0, t)),
            cb_spec, cb_spec, cb_spec,
        ],
        out_specs=[
            pl.BlockSpec((1, C, _T_BLK), lambda b, t: (b, 0, t)),
            pl.BlockSpec((8, _T_BLK), lambda b, t: (0, b * n_tblk + t)),
            pl.BlockSpec((1, 1), lambda b, t: (0, 0)),
        ],
        out_shape=[
            jax.ShapeDtypeStruct((B, C, T), jnp.float32),
            jax.ShapeDtypeStruct((8, B * T), jnp.int32),
            jax.ShapeDtypeStruct((1, 1), jnp.float32),
        ],
    )(xin, codebooks, cb_hi, cb_lo)
    return q, loss.reshape(()), idx8[:2 * _LAYERS]
